# baseline (speedup 1.0000x reference)
import jax
import jax.numpy as jnp
from jax.experimental import pallas as pl
from jax.experimental.pallas import tpu as pltpu

B = 4
N = 8192
F = 1024
BLK = 512


def _body(t_ref, o_ref, sem):
    i = pl.program_id(0)
    for b in range(B):
        pltpu.make_async_copy(
            t_ref, o_ref.at[b, pl.ds(i * BLK, BLK), :], sem.at[b]
        ).start()
    for b in range(B):
        pltpu.make_async_copy(
            t_ref, o_ref.at[b, pl.ds(i * BLK, BLK), :], sem.at[b]
        ).wait()


def kernel(batch_size, table):
    del batch_size
    return pl.pallas_call(
        _body,
        grid=(N // BLK,),
        in_specs=[pl.BlockSpec((BLK, F), lambda i: (i, 0))],
        out_specs=pl.BlockSpec(memory_space=pl.ANY),
        out_shape=jax.ShapeDtypeStruct((B, N, F), jnp.float32),
        scratch_shapes=[pltpu.SemaphoreType.DMA((B,))],
    )(table)


# speedup vs baseline: 1.0732x; 1.0732x over previous
import jax
import jax.numpy as jnp
from jax.experimental import pallas as pl

B = 4
N = 8192
F = 1024
BLK = 256


def _body(t_ref, o_ref):
    o_ref[...] = jnp.broadcast_to(t_ref[...][None], (B, BLK, F))


def kernel(batch_size, table):
    del batch_size
    return pl.pallas_call(
        _body,
        grid=(N // BLK,),
        in_specs=[pl.BlockSpec((BLK, F), lambda i: (i, 0))],
        out_specs=pl.BlockSpec((B, BLK, F), lambda i: (0, i, 0)),
        out_shape=jax.ShapeDtypeStruct((B, N, F), jnp.float32),
    )(table)


# speedup vs baseline: 1.1712x; 1.0913x over previous
import jax
import jax.numpy as jnp
from jax.experimental import pallas as pl

B = 4
N = 8192
F = 1024
BLK = 1024


def _body(t_ref, o_ref):
    o_ref[...] = jnp.broadcast_to(t_ref[...][None], (B, BLK, F))


def kernel(batch_size, table):
    del batch_size
    return pl.pallas_call(
        _body,
        grid=(N // BLK,),
        in_specs=[pl.BlockSpec((BLK, F), lambda i: (i, 0))],
        out_specs=pl.BlockSpec((B, BLK, F), lambda i: (0, i, 0)),
        out_shape=jax.ShapeDtypeStruct((B, N, F), jnp.float32),
    )(table)
